# Initial kernel scaffold; baseline (speedup 1.0000x reference)
import jax
import jax.numpy as jnp
from jax.experimental import pallas as pl


def kernel(positions, path_points, arclengths, tangents, normals, left_widths, right_widths):
    raise NotImplementedError("write your pallas kernel here")


# speedup vs baseline: 2.6049x; 2.6049x over previous
import jax
import jax.numpy as jnp
from jax.experimental import pallas as pl

_QB = 128


def _argmin_body(pos_ref, pathT_ref, idx_ref):
    pos = pos_ref[...]
    pathT = pathT_ref[...]
    m = pathT.shape[1]
    dot = jax.lax.dot_general(pos, pathT, (((1,), (0,)), ((), ())),
                              preferred_element_type=jnp.float32)
    pn = jnp.sum(pos * pos, axis=-1, keepdims=True)
    qn = jnp.sum(pathT * pathT, axis=0, keepdims=True)
    d2 = pn + qn - 2.0 * dot
    best = jnp.min(d2, axis=1, keepdims=True)
    iota = jax.lax.broadcasted_iota(jnp.int32, d2.shape, 1)
    idx_ref[...] = jnp.min(jnp.where(d2 == best, iota, jnp.int32(m)), axis=1)


def _nearest_idx(positions, path_points, interpret=False):
    q = positions.shape[0]
    m = path_points.shape[0]
    pathT = path_points.T
    return pl.pallas_call(
        _argmin_body,
        grid=(q // _QB,),
        in_specs=[
            pl.BlockSpec((_QB, 2), lambda i: (i, 0)),
            pl.BlockSpec((2, m), lambda i: (0, 0)),
        ],
        out_specs=pl.BlockSpec((_QB,), lambda i: (i,)),
        out_shape=jax.ShapeDtypeStruct((q,), jnp.int32),
        interpret=interpret,
    )(positions, pathT)


def kernel(positions, path_points, arclengths, tangents, normals, left_widths, right_widths):
    idx = _nearest_idx(positions, path_points)
    closest_point_r = jnp.take(arclengths, idx)
    closest_point_values = jnp.take(path_points, idx, axis=0)
    closest_point_tangents = jnp.take(tangents, idx, axis=0)
    closest_point_normals = jnp.take(normals, idx, axis=0)
    deltas = positions - closest_point_values
    normal_projections = jnp.sum(deltas * closest_point_normals, axis=-1)
    left_width_vals = jnp.take(left_widths, idx)
    right_width_vals = jnp.take(right_widths, idx)
    return (closest_point_r, closest_point_values, closest_point_tangents, closest_point_normals, deltas, normal_projections, left_width_vals, right_width_vals)
